# baseline (speedup 1.0000x reference)
import functools

import jax
import jax.numpy as jnp
from jax import lax
from jax.experimental import pallas as pl
from jax.experimental.pallas import tpu as pltpu

_LANE = 128


def _round_up(n, m):
    return ((n + m - 1) // m) * m


def _manual_kernel(x_ref, w_ref, b_ref, o_ref, bufs, sems, *,
                   ch_rows, n_chunks, length, d_in, depth):
    core = pl.program_id(0)
    base = core * (n_chunks * ch_rows)
    w = w_ref[...]
    b = b_ref[...]
    d_out = w_ref.shape[-1]

    def copy(k):
        return pltpu.make_async_copy(
            x_ref.at[pl.ds(base + k * ch_rows, ch_rows)],
            bufs.at[k % depth],
            sems.at[k % depth])

    for k in range(min(depth, n_chunks)):
        copy(k).start()

    for k in range(n_chunks):
        copy(k).wait()
        xb = bufs[k % depth].astype(jnp.bfloat16)
        h = lax.dot_general(
            xb.reshape(ch_rows * length, d_in), w,
            dimension_numbers=(((1,), (0,)), ((), ())),
            preferred_element_type=jnp.float32)
        cm = jnp.max(h.reshape(ch_rows, length, d_out), axis=1)
        o_ref[pl.ds(k * ch_rows, ch_rows), :] = jnp.maximum(
            cm + b, 0.0).astype(o_ref.dtype)
        if k + depth < n_chunks:
            copy(k + depth).start()


def _manual_fused(x, weight, bias, wp, bp, *, ch_rows, depth):
    B, L, d_in = x.shape
    d_out = weight.shape[1]
    dpo = wp.shape[1]
    half = B // 2
    n_chunks = half // ch_rows
    out = pl.pallas_call(
        functools.partial(_manual_kernel, ch_rows=ch_rows, n_chunks=n_chunks,
                          length=L, d_in=d_in, depth=depth),
        out_shape=jax.ShapeDtypeStruct((B, dpo), x.dtype),
        grid=(2,),
        in_specs=[
            pl.BlockSpec(memory_space=pl.ANY),
            pl.BlockSpec((d_in, dpo), lambda c: (0, 0)),
            pl.BlockSpec((1, dpo), lambda c: (0, 0)),
        ],
        out_specs=pl.BlockSpec((half, dpo), lambda c: (c, 0)),
        scratch_shapes=[
            pltpu.VMEM((depth, ch_rows, L, d_in), x.dtype),
            pltpu.SemaphoreType.DMA((depth,)),
        ],
        compiler_params=pltpu.CompilerParams(
            dimension_semantics=("parallel",),
            vmem_limit_bytes=64 * 1024 * 1024,
        ),
    )(x, wp, bp)
    return out


def _fused_kernel(*refs, tb, l_chunk, rows_per_dot, n_streams):
    x_refs = refs[:n_streams]
    w_ref, b_ref, o_ref = refs[n_streams:]
    d_in = x_refs[0].shape[-1]
    d_out = w_ref.shape[-1]
    w = w_ref[...]
    b = b_ref[...]
    for b0 in range(0, tb, rows_per_dot):
        cm = None
        for x_ref in x_refs:
            xb = x_ref[pl.ds(b0, rows_per_dot), 0, :, :].astype(jnp.bfloat16)
            h = lax.dot_general(
                xb.reshape(rows_per_dot * l_chunk, d_in), w,
                dimension_numbers=(((1,), (0,)), ((), ())),
                preferred_element_type=jnp.float32)
            sm = jnp.max(h.reshape(rows_per_dot, l_chunk, d_out), axis=1)
            cm = sm if cm is None else jnp.maximum(cm, sm)
        o_ref[pl.ds(b0, rows_per_dot), :] = jnp.maximum(
            cm + b, 0.0).astype(o_ref.dtype)


def _emitter_fused(x, weight, bias, wp, bp, *, tb=64, rows_per_dot=16,
                   n_streams=2):
    B, L, d_in = x.shape
    dpo = wp.shape[1]

    if L % n_streams != 0:
        n_streams = 1
    l_chunk = L // n_streams
    xs = x.reshape(B, n_streams, l_chunk, d_in)

    tb = min(tb, B)
    rows_per_dot = min(rows_per_dot, tb)
    while rows_per_dot > 1 and tb % rows_per_dot != 0:
        rows_per_dot //= 2
    nb = pl.cdiv(B, tb)

    def _x_spec(si):
        return pl.BlockSpec((tb, 1, l_chunk, d_in),
                            lambda bi, si=si: (bi, si, 0, 0))

    return pl.pallas_call(
        functools.partial(_fused_kernel, tb=tb, l_chunk=l_chunk,
                          rows_per_dot=rows_per_dot, n_streams=n_streams),
        out_shape=jax.ShapeDtypeStruct((B, dpo), x.dtype),
        grid=(nb,),
        in_specs=[_x_spec(si) for si in range(n_streams)] + [
            pl.BlockSpec((d_in, dpo), lambda bi: (0, 0)),
            pl.BlockSpec((1, dpo), lambda bi: (0, 0)),
        ],
        out_specs=pl.BlockSpec((tb, dpo), lambda bi: (bi, 0)),
        compiler_params=pltpu.CompilerParams(
            dimension_semantics=("parallel",),
            vmem_limit_bytes=64 * 1024 * 1024,
        ),
    )(*([xs] * n_streams), wp, bp)


def kernel(x, weight, bias, *, ch_rows=16, depth=4):
    B, L, d_in = x.shape
    d_out = weight.shape[1]

    dpo = _round_up(d_out, _LANE)
    wp = jnp.pad(weight, ((0, 0), (0, dpo - d_out))).astype(jnp.bfloat16)
    bp = jnp.pad(bias.reshape(1, -1).astype(jnp.float32),
                 ((0, 0), (0, dpo - d_out)))

    if B % (2 * ch_rows) == 0 and L % 8 == 0:
        out = _manual_fused(x, weight, bias, wp, bp,
                            ch_rows=ch_rows, depth=depth)
    else:
        out = _emitter_fused(x, weight, bias, wp, bp)
    if dpo != d_out:
        out = out[:, :d_out]
    return out


# speedup vs baseline: 1.0257x; 1.0257x over previous
import functools

import jax
import jax.numpy as jnp
from jax import lax
from jax.experimental import pallas as pl
from jax.experimental.pallas import tpu as pltpu

_LANE = 128


def _round_up(n, m):
    return ((n + m - 1) // m) * m


def _manual_kernel(x_ref, w_ref, b_ref, o_ref, bufs, sems, *,
                   ch_rows, n_chunks, length, d_in, depth):
    core = pl.program_id(0)
    base = core * (n_chunks * ch_rows)
    w = w_ref[...]
    b = b_ref[...]
    d_out = w_ref.shape[-1]

    def copy(k):
        return pltpu.make_async_copy(
            x_ref.at[pl.ds(base + k * ch_rows, ch_rows)],
            bufs.at[k % depth],
            sems.at[k % depth])

    for k in range(min(depth, n_chunks)):
        copy(k).start()

    for k in range(n_chunks):
        copy(k).wait()
        xb = bufs[k % depth].astype(jnp.bfloat16)
        h = lax.dot_general(
            xb.reshape(ch_rows * length, d_in), w,
            dimension_numbers=(((1,), (0,)), ((), ())),
            preferred_element_type=jnp.float32)
        cm = jnp.max(h.reshape(ch_rows, length, d_out), axis=1)
        o_ref[pl.ds(k * ch_rows, ch_rows), :] = jnp.maximum(
            cm + b, 0.0).astype(o_ref.dtype)
        if k + depth < n_chunks:
            copy(k + depth).start()


def _manual_fused(x, weight, bias, wp, bp, *, ch_rows, depth):
    B, L, d_in = x.shape
    d_out = weight.shape[1]
    dpo = wp.shape[1]
    half = B // 2
    n_chunks = half // ch_rows
    out = pl.pallas_call(
        functools.partial(_manual_kernel, ch_rows=ch_rows, n_chunks=n_chunks,
                          length=L, d_in=d_in, depth=depth),
        out_shape=jax.ShapeDtypeStruct((B, dpo), x.dtype),
        grid=(2,),
        in_specs=[
            pl.BlockSpec(memory_space=pl.ANY),
            pl.BlockSpec((d_in, dpo), lambda c: (0, 0)),
            pl.BlockSpec((1, dpo), lambda c: (0, 0)),
        ],
        out_specs=pl.BlockSpec((half, dpo), lambda c: (c, 0)),
        scratch_shapes=[
            pltpu.VMEM((depth, ch_rows, L, d_in), x.dtype),
            pltpu.SemaphoreType.DMA((depth,)),
        ],
        compiler_params=pltpu.CompilerParams(
            dimension_semantics=("parallel",),
            vmem_limit_bytes=64 * 1024 * 1024,
        ),
    )(x, wp, bp)
    return out


def _fused_kernel(*refs, tb, l_chunk, rows_per_dot, n_streams):
    x_refs = refs[:n_streams]
    w_ref, b_ref, o_ref = refs[n_streams:]
    d_in = x_refs[0].shape[-1]
    d_out = w_ref.shape[-1]
    w = w_ref[...]
    b = b_ref[...]
    for b0 in range(0, tb, rows_per_dot):
        cm = None
        for x_ref in x_refs:
            xb = x_ref[pl.ds(b0, rows_per_dot), 0, :, :].astype(jnp.bfloat16)
            h = lax.dot_general(
                xb.reshape(rows_per_dot * l_chunk, d_in), w,
                dimension_numbers=(((1,), (0,)), ((), ())),
                preferred_element_type=jnp.float32)
            sm = jnp.max(h.reshape(rows_per_dot, l_chunk, d_out), axis=1)
            cm = sm if cm is None else jnp.maximum(cm, sm)
        o_ref[pl.ds(b0, rows_per_dot), :] = jnp.maximum(
            cm + b, 0.0).astype(o_ref.dtype)


def _emitter_fused(x, weight, bias, wp, bp, *, tb=64, rows_per_dot=16,
                   n_streams=2):
    B, L, d_in = x.shape
    dpo = wp.shape[1]

    if L % n_streams != 0:
        n_streams = 1
    l_chunk = L // n_streams
    xs = x.reshape(B, n_streams, l_chunk, d_in)

    tb = min(tb, B)
    rows_per_dot = min(rows_per_dot, tb)
    while rows_per_dot > 1 and tb % rows_per_dot != 0:
        rows_per_dot //= 2
    nb = pl.cdiv(B, tb)

    def _x_spec(si):
        return pl.BlockSpec((tb, 1, l_chunk, d_in),
                            lambda bi, si=si: (bi, si, 0, 0))

    return pl.pallas_call(
        functools.partial(_fused_kernel, tb=tb, l_chunk=l_chunk,
                          rows_per_dot=rows_per_dot, n_streams=n_streams),
        out_shape=jax.ShapeDtypeStruct((B, dpo), x.dtype),
        grid=(nb,),
        in_specs=[_x_spec(si) for si in range(n_streams)] + [
            pl.BlockSpec((d_in, dpo), lambda bi: (0, 0)),
            pl.BlockSpec((1, dpo), lambda bi: (0, 0)),
        ],
        out_specs=pl.BlockSpec((tb, dpo), lambda bi: (bi, 0)),
        compiler_params=pltpu.CompilerParams(
            dimension_semantics=("parallel",),
            vmem_limit_bytes=64 * 1024 * 1024,
        ),
    )(*([xs] * n_streams), wp, bp)


def kernel(x, weight, bias, *, ch_rows=32, depth=3):
    B, L, d_in = x.shape
    d_out = weight.shape[1]

    dpo = _round_up(d_out, _LANE)
    wp = jnp.pad(weight, ((0, 0), (0, dpo - d_out))).astype(jnp.bfloat16)
    bp = jnp.pad(bias.reshape(1, -1).astype(jnp.float32),
                 ((0, 0), (0, dpo - d_out)))

    if B % (2 * ch_rows) == 0 and L % 8 == 0:
        out = _manual_fused(x, weight, bias, wp, bp,
                            ch_rows=ch_rows, depth=depth)
    else:
        out = _emitter_fused(x, weight, bias, wp, bp)
    if dpo != d_out:
        out = out[:, :d_out]
    return out


# speedup vs baseline: 1.0280x; 1.0023x over previous
import functools

import jax
import jax.numpy as jnp
from jax import lax
from jax.experimental import pallas as pl
from jax.experimental.pallas import tpu as pltpu

_LANE = 128


def _round_up(n, m):
    return ((n + m - 1) // m) * m


def _fused_kernel(*refs, tb, l_chunk, rows_per_dot, n_streams):
    x_refs = refs[:n_streams]
    w_ref, b_ref, o_ref = refs[n_streams:]
    d_in = x_refs[0].shape[-1]
    d_out = w_ref.shape[-1]
    w = w_ref[...]
    b = b_ref[...]
    for b0 in range(0, tb, rows_per_dot):
        cm = None
        for x_ref in x_refs:
            xb = x_ref[pl.ds(b0, rows_per_dot), 0, :, :].astype(jnp.bfloat16)
            h = lax.dot_general(
                xb.reshape(rows_per_dot * l_chunk, d_in), w,
                dimension_numbers=(((1,), (0,)), ((), ())),
                preferred_element_type=jnp.float32)
            sm = jnp.max(h.reshape(rows_per_dot, l_chunk, d_out), axis=1)
            cm = sm if cm is None else jnp.maximum(cm, sm)
        o_ref[pl.ds(b0, rows_per_dot), :] = jnp.maximum(
            cm + b, 0.0).astype(o_ref.dtype)


def _fused_linear_relu_maxpool(x, weight, bias, *, tb=64, rows_per_dot=16,
                               n_streams=2):
    B, L, d_in = x.shape
    d_out = weight.shape[1]
    out_dtype = x.dtype

    if L % n_streams != 0:
        n_streams = 1
    l_chunk = L // n_streams
    xs = x.reshape(B, n_streams, l_chunk, d_in)

    dpo = _round_up(d_out, _LANE)
    wp = jnp.pad(weight, ((0, 0), (0, dpo - d_out))).astype(jnp.bfloat16)
    bp = jnp.pad(bias.reshape(1, -1).astype(jnp.float32),
                 ((0, 0), (0, dpo - d_out)))

    tb = min(tb, B)
    rows_per_dot = min(rows_per_dot, tb)
    while rows_per_dot > 1 and tb % rows_per_dot != 0:
        rows_per_dot //= 2
    nb = pl.cdiv(B, tb)

    def _x_spec(si):
        return pl.BlockSpec((tb, 1, l_chunk, d_in),
                            lambda bi, si=si: (bi, si, 0, 0))

    out = pl.pallas_call(
        functools.partial(_fused_kernel, tb=tb, l_chunk=l_chunk,
                          rows_per_dot=rows_per_dot, n_streams=n_streams),
        out_shape=jax.ShapeDtypeStruct((B, dpo), out_dtype),
        grid=(nb,),
        in_specs=[_x_spec(si) for si in range(n_streams)] + [
            pl.BlockSpec((d_in, dpo), lambda bi: (0, 0)),
            pl.BlockSpec((1, dpo), lambda bi: (0, 0)),
        ],
        out_specs=pl.BlockSpec((tb, dpo), lambda bi: (bi, 0)),
        compiler_params=pltpu.CompilerParams(
            dimension_semantics=("parallel",),
            vmem_limit_bytes=64 * 1024 * 1024,
        ),
    )(*([xs] * n_streams), wp, bp)
    if dpo != d_out:
        out = out[:, :d_out]
    return out


def kernel(x, weight, bias):
    return _fused_linear_relu_maxpool(x, weight, bias)
